# baseline (speedup 1.0000x reference)
import functools

import jax
import jax.numpy as jnp
from jax import lax
from jax.experimental import pallas as pl
from jax.experimental.pallas import tpu as pltpu
from jax.experimental.pallas import tpu_sc as plsc

D = 1024
I = 512
SI = 1024
E = 16
EP = 128
T = 4096
K = 2
R = T * K

TMA = 512
NTA = T // TMA
TMG = 512
NTG = R // TMG
NSTEPS = NTG + E - 1
NW = 32
CHUNK = T // NW

NEG = -1e30
WFLOOR = 1e-30


def _lane_val(vec, idx, lanerow):
    return jnp.sum(jnp.where(lanerow == idx, vec, 0))


def _router_shared_body(x32_ref, xb_ref, gate_ref, ws1_ref, ws2_ref,
                        coeff_ref, rank_ref, shared_ref,
                        offs_ref, sm_ref, se_ref, sv_ref, si_ref,
                        cnt_ref):
    t = pl.program_id(0)

    x32 = x32_ref[...]
    logits = jnp.dot(x32, gate_ref[...], preferred_element_type=jnp.float32)
    lane = jax.lax.broadcasted_iota(jnp.int32, logits.shape, 1)
    valid = lane < E
    logits = jnp.where(valid, logits, NEG)
    m = jnp.max(logits, axis=1, keepdims=True)
    p = jnp.exp(logits - m)
    p = jnp.where(valid, p, 0.0)
    p = p / jnp.sum(p, axis=1, keepdims=True)
    p1 = jnp.max(p, axis=1, keepdims=True)
    i1 = jnp.min(jnp.where(p == p1, lane, EP), axis=1, keepdims=True)
    mask1 = lane == i1
    pm = jnp.where(mask1, -1.0, p)
    p2 = jnp.max(pm, axis=1, keepdims=True)
    i2 = jnp.min(jnp.where(pm == p2, lane, EP), axis=1, keepdims=True)
    denom = p1 + p2
    coeff = (jnp.where(mask1, p1, 0.0)
             + jnp.where(lane == i2, p2, 0.0)) / denom
    active = mask1 | (lane == i2)
    coeff = jnp.where(active, jnp.maximum(coeff, WFLOOR), 0.0)
    coeff_ref[...] = coeff[:, :E]

    @pl.when(t == 0)
    def _init_cnt():
        cnt_ref[...] = jnp.zeros((1, EP), jnp.int32)

    act_i = active.astype(jnp.int32)
    s = act_i
    sh = 1
    while sh < TMA:
        z = jnp.zeros((sh, EP), jnp.int32)
        s = s + jnp.concatenate([z, s[:-sh]], axis=0)
        sh *= 2
    excl = s - act_i
    carry = cnt_ref[...]
    rank_ref[...] = (carry + excl)[:, :E]
    new_cnt = carry + jnp.sum(act_i, axis=0, keepdims=True)
    cnt_ref[...] = new_cnt

    xb = xb_ref[...]
    h = jnp.dot(xb, ws1_ref[...], preferred_element_type=jnp.float32)
    act = (jax.nn.silu(h[:, :SI]) * h[:, SI:]).astype(jnp.bfloat16)
    shared_ref[...] = jnp.dot(act, ws2_ref[...],
                              preferred_element_type=jnp.float32)

    @pl.when(t == NTA - 1)
    def _meta():
        lanerow = jax.lax.broadcasted_iota(jnp.int32, (1, EP), 1)
        tot = new_cnt
        o = tot
        shl = 1
        while shl < 32:
            o = o + jnp.concatenate(
                [jnp.zeros((1, shl), jnp.int32), o[:, :-shl]], axis=1)
            shl *= 2
        offs = o - tot
        offs_ref[...] = offs

        e_lo = jnp.zeros((1, EP), jnp.int32)
        e_hi = jnp.zeros((1, EP), jnp.int32)
        for e in range(E):
            off_e = _lane_val(offs, e, lanerow)
            off_e1 = _lane_val(offs, e + 1, lanerow)
            e_lo = e_lo + (off_e1 <= lanerow * TMG).astype(jnp.int32)
            e_hi = e_hi + (off_e < (lanerow + 1) * TMG).astype(jnp.int32)
        e_hi = e_hi - 1
        n = jnp.where(lanerow < NTG, e_hi - e_lo + 1, 0)
        si_ = n
        shl = 1
        while shl < 32:
            si_ = si_ + jnp.concatenate(
                [jnp.zeros((1, shl), jnp.int32), si_[:, :-shl]], axis=1)
            shl *= 2
        start = si_ - n
        total_steps = (_lane_val(start, NTG - 1, lanerow)
                       + _lane_val(n, NTG - 1, lanerow))
        m_of_i = jnp.full((1, EP), -1, jnp.int32)
        for mm in range(NTG):
            s_m = _lane_val(start, mm, lanerow)
            m_of_i = m_of_i + (s_m <= lanerow).astype(jnp.int32)
        m_of_i = jnp.clip(m_of_i, 0, NTG - 1)
        s_at = jnp.zeros((1, EP), jnp.int32)
        elo_at = jnp.zeros((1, EP), jnp.int32)
        for mm in range(NTG):
            cond = m_of_i == mm
            s_at = jnp.where(cond, _lane_val(start, mm, lanerow), s_at)
            elo_at = jnp.where(cond, _lane_val(e_lo, mm, lanerow), elo_at)
        e_of_i = jnp.clip(elo_at + lanerow - s_at, 0, E - 1)
        valid_i = (lanerow < total_steps).astype(jnp.int32)
        init_i = ((lanerow == s_at).astype(jnp.int32)) * valid_i
        sm_ref[...] = m_of_i
        se_ref[...] = e_of_i
        sv_ref[...] = valid_i
        si_ref[...] = init_i


def _gmm_body(sm_ref, se_ref, sv_ref, si_ref, off_ref,
              xs_ref, w1_ref, w2_ref, y_ref):
    i = pl.program_id(0)

    @pl.when(si_ref[i] == 1)
    def _zero():
        y_ref[...] = jnp.zeros_like(y_ref)

    @pl.when(sv_ref[i] == 1)
    def _compute():
        e = se_ref[i]
        lo = off_ref[e]
        hi = off_ref[e + 1]
        xsb = xs_ref[...].astype(jnp.bfloat16)
        h = jnp.dot(xsb, w1_ref[0], preferred_element_type=jnp.float32)
        act = (jax.nn.silu(h[:, :I]) * h[:, I:]).astype(jnp.bfloat16)
        yv = jnp.dot(act, w2_ref[0], preferred_element_type=jnp.float32)
        rows = (sm_ref[i] * TMG
                + jax.lax.broadcasted_iota(jnp.int32, (TMG, 1), 0))
        msk = (rows >= lo) & (rows < hi)
        y_ref[...] += jnp.where(msk, yv, 0.0)


def _make_dispatch():
    mesh = plsc.VectorSubcoreMesh(core_axis_name="c", subcore_axis_name="s")

    @functools.partial(
        pl.kernel,
        out_type=[
            jax.ShapeDtypeStruct((R, D), jnp.float32),
            jax.ShapeDtypeStruct((NW * 2, 64), jnp.int32),
            jax.ShapeDtypeStruct((NW * 2, 64), jnp.int32),
            jax.ShapeDtypeStruct((NW * 2, 64), jnp.float32),
            jax.ShapeDtypeStruct((NW * 2, 64), jnp.float32),
        ],
        mesh=mesh,
        scratch_types=[
            pltpu.VMEM((CHUNK, E), jnp.float32),
            pltpu.VMEM((CHUNK, E), jnp.int32),
            pltpu.VMEM((16,), jnp.int32),
            pltpu.VMEM((2, 64), jnp.int32),
            pltpu.VMEM((2, 64), jnp.int32),
            pltpu.VMEM((2, 64), jnp.float32),
            pltpu.VMEM((2, 64), jnp.float32),
            pltpu.VMEM((64, D), jnp.float32),
            pltpu.SemaphoreType.DMA,
        ],
        compiler_params=pltpu.CompilerParams(
            needs_layout_passes=False, use_tc_tiling_on_sc=False),
    )
    def dispatch(x_hbm, coeff_hbm, rank_hbm, offs_hbm,
                 xs_hbm, d0_hbm, d1_hbm, w0_hbm, w1_hbm,
                 cbuf, rbuf, offs_v, d0buf, d1buf, w0buf, w1buf, xbuf, sem):
        wid = lax.axis_index("s") * 2 + lax.axis_index("c")
        base = wid * CHUNK
        pltpu.sync_copy(coeff_hbm.at[pl.ds(base, CHUNK)], cbuf)
        pltpu.sync_copy(rank_hbm.at[pl.ds(base, CHUNK)], rbuf)
        pltpu.sync_copy(offs_hbm.at[pl.ds(0, 16)], offs_v)
        lanes = lax.iota(jnp.int32, 16)
        offs_vec = offs_v[...]
        for g in range(8):
            d0a = jnp.zeros((16,), jnp.int32)
            d1a = jnp.zeros((16,), jnp.int32)
            w0a = jnp.zeros((16,), jnp.float32)
            w1a = jnp.zeros((16,), jnp.float32)
            for j in range(16):
                tj = g * 16 + j
                v = cbuf[tj, :]
                r = rbuf[tj, :]
                act = v > 0.0
                e0 = jnp.min(jnp.where(act, lanes, E))
                e1 = jnp.min(jnp.where(jnp.logical_and(act, lanes > e0),
                                       lanes, E))
                dd = r + offs_vec
                sel0 = lanes == e0
                sel1 = lanes == e1
                d0s = jnp.sum(jnp.where(sel0, dd, 0))
                d1s = jnp.sum(jnp.where(sel1, dd, 0))
                w0s = jnp.sum(jnp.where(sel0, v, 0.0))
                w1s = jnp.sum(jnp.where(sel1, v, 0.0))
                here = lanes == j
                d0a = jnp.where(here, d0s, d0a)
                d1a = jnp.where(here, d1s, d1a)
                w0a = jnp.where(here, w0s, w0a)
                w1a = jnp.where(here, w1s, w1a)
            sr, hc = g // 4, (g % 4) * 16
            d0buf[sr, pl.ds(hc, 16)] = d0a
            d1buf[sr, pl.ds(hc, 16)] = d1a
            w0buf[sr, pl.ds(hc, 16)] = w0a
            w1buf[sr, pl.ds(hc, 16)] = w1a
        pltpu.sync_copy(d0buf, d0_hbm.at[pl.ds(2 * wid, 2)])
        pltpu.sync_copy(d1buf, d1_hbm.at[pl.ds(2 * wid, 2)])
        pltpu.sync_copy(w0buf, w0_hbm.at[pl.ds(2 * wid, 2)])
        pltpu.sync_copy(w1buf, w1_hbm.at[pl.ds(2 * wid, 2)])
        for sr in range(2):
            pltpu.sync_copy(x_hbm.at[pl.ds(base + 64 * sr, 64)], xbuf)
            pltpu.async_copy(xbuf, xs_hbm.at[d0buf.at[sr]], sem).wait()
            pltpu.async_copy(xbuf, xs_hbm.at[d1buf.at[sr]], sem).wait()

    return dispatch


def _make_combine():
    mesh = plsc.VectorSubcoreMesh(core_axis_name="c", subcore_axis_name="s")

    @functools.partial(
        pl.kernel,
        out_type=jax.ShapeDtypeStruct((T, D), jnp.float32),
        mesh=mesh,
        scratch_types=[
            pltpu.VMEM((2, 64), jnp.int32),
            pltpu.VMEM((2, 64), jnp.int32),
            pltpu.VMEM((2, 64), jnp.float32),
            pltpu.VMEM((2, 64), jnp.float32),
            pltpu.VMEM((32, D), jnp.float32),
            pltpu.VMEM((32, D), jnp.float32),
            pltpu.VMEM((32, D), jnp.float32),
            pltpu.SemaphoreType.DMA,
        ],
        compiler_params=pltpu.CompilerParams(
            needs_layout_passes=False, use_tc_tiling_on_sc=False),
    )
    def combine(y_hbm, d0_hbm, d1_hbm, w0_hbm, w1_hbm, shared_hbm, out_hbm,
                d0v, d1v, w0v, w1v, y0buf, y1buf, sbuf, sem):
        wid = lax.axis_index("s") * 2 + lax.axis_index("c")
        base = wid * CHUNK
        pltpu.sync_copy(d0_hbm.at[pl.ds(2 * wid, 2)], d0v)
        pltpu.sync_copy(d1_hbm.at[pl.ds(2 * wid, 2)], d1v)
        pltpu.sync_copy(w0_hbm.at[pl.ds(2 * wid, 2)], w0v)
        pltpu.sync_copy(w1_hbm.at[pl.ds(2 * wid, 2)], w1v)
        lanes = lax.iota(jnp.int32, 16)
        for q in range(4):
            tb = base + 32 * q
            idx0 = d0v.at[q // 2, pl.ds((q % 2) * 32, 32)]
            idx1 = d1v.at[q // 2, pl.ds((q % 2) * 32, 32)]
            g0 = pltpu.async_copy(y_hbm.at[idx0], y0buf, sem)
            g1 = pltpu.async_copy(y_hbm.at[idx1], y1buf, sem)
            pltpu.sync_copy(shared_hbm.at[pl.ds(tb, 32)], sbuf)
            g0.wait()
            g1.wait()

            def body(j, carry):
                hw = (j // 16) * 16
                w0g = w0v[q // 2, pl.ds((q % 2) * 32 + hw, 16)]
                w1g = w1v[q // 2, pl.ds((q % 2) * 32 + hw, 16)]
                jm = j % 16
                w0s = jnp.sum(jnp.where(lanes == jm, w0g, 0.0))
                w1s = jnp.sum(jnp.where(lanes == jm, w1g, 0.0))
                for kk in range(D // 16):
                    sl = pl.ds(kk * 16, 16)
                    sbuf[j, sl] = (sbuf[j, sl]
                                   + w0s * y0buf[j, sl]
                                   + w1s * y1buf[j, sl])
                return carry

            lax.fori_loop(0, 32, body, 0)
            pltpu.sync_copy(sbuf, out_hbm.at[pl.ds(tb, 32)])

    return combine


def _moe(x, gate_pad, w1b, w2b, ws1b, ws2b, interpret=False):
    xb = x.astype(jnp.bfloat16)

    outs = pl.pallas_call(
        _router_shared_body,
        grid=(NTA,),
        in_specs=[
            pl.BlockSpec((TMA, D), lambda t: (t, 0)),
            pl.BlockSpec((TMA, D), lambda t: (t, 0)),
            pl.BlockSpec((D, EP), lambda t: (0, 0)),
            pl.BlockSpec((D, 2 * SI), lambda t: (0, 0)),
            pl.BlockSpec((SI, D), lambda t: (0, 0)),
        ],
        out_specs=[
            pl.BlockSpec((TMA, E), lambda t: (t, 0)),
            pl.BlockSpec((TMA, E), lambda t: (t, 0)),
            pl.BlockSpec((TMA, D), lambda t: (t, 0)),
            pl.BlockSpec((1, EP), lambda t: (0, 0)),
            pl.BlockSpec((1, EP), lambda t: (0, 0)),
            pl.BlockSpec((1, EP), lambda t: (0, 0)),
            pl.BlockSpec((1, EP), lambda t: (0, 0)),
            pl.BlockSpec((1, EP), lambda t: (0, 0)),
        ],
        out_shape=[
            jax.ShapeDtypeStruct((T, E), jnp.float32),
            jax.ShapeDtypeStruct((T, E), jnp.int32),
            jax.ShapeDtypeStruct((T, D), jnp.float32),
            jax.ShapeDtypeStruct((1, EP), jnp.int32),
            jax.ShapeDtypeStruct((1, EP), jnp.int32),
            jax.ShapeDtypeStruct((1, EP), jnp.int32),
            jax.ShapeDtypeStruct((1, EP), jnp.int32),
            jax.ShapeDtypeStruct((1, EP), jnp.int32),
        ],
        scratch_shapes=[pltpu.VMEM((1, EP), jnp.int32)],
        compiler_params=pltpu.CompilerParams(
            dimension_semantics=("arbitrary",)),
        interpret=interpret,
    )(x, xb, gate_pad, ws1b, ws2b)
    coeff16, rank16, shared, offs, sm, se, sv, si = outs
    offs1 = offs.reshape(EP)
    sm1 = sm.reshape(EP)
    se1 = se.reshape(EP)
    sv1 = sv.reshape(EP)
    si1 = si.reshape(EP)

    dispatch = _make_dispatch()
    xs, d0, d1, w0, w1 = dispatch(x, coeff16, rank16, offs1)

    grid_spec = pltpu.PrefetchScalarGridSpec(
        num_scalar_prefetch=5,
        grid=(NSTEPS,),
        in_specs=[
            pl.BlockSpec((TMG, D), lambda i, sm, se, sv, si, off: (sm[i], 0)),
            pl.BlockSpec((1, D, 2 * I),
                         lambda i, sm, se, sv, si, off: (se[i], 0, 0)),
            pl.BlockSpec((1, I, D),
                         lambda i, sm, se, sv, si, off: (se[i], 0, 0)),
        ],
        out_specs=pl.BlockSpec((TMG, D),
                               lambda i, sm, se, sv, si, off: (sm[i], 0)),
    )
    y = pl.pallas_call(
        _gmm_body,
        grid_spec=grid_spec,
        out_shape=jax.ShapeDtypeStruct((R, D), jnp.float32),
        compiler_params=pltpu.CompilerParams(
            dimension_semantics=("arbitrary",)),
        interpret=interpret,
    )(sm1, se1, sv1, si1, offs1, xs, w1b, w2b)

    combine = _make_combine()
    return combine(y, d0, d1, w0, w1, shared)


def kernel(hidden_states, gate_w, w_gate_up, w_down, ws_gate_up, ws_down):
    orig_shape = hidden_states.shape
    x = hidden_states.reshape(-1, D)
    gate_pad = jnp.pad(gate_w, ((0, 0), (0, EP - E)))
    w1b = w_gate_up.astype(jnp.bfloat16)
    w2b = w_down.astype(jnp.bfloat16)
    ws1b = ws_gate_up.astype(jnp.bfloat16)
    ws2b = ws_down.astype(jnp.bfloat16)
    out = _moe(x, gate_pad, w1b, w2b, ws1b, ws2b)
    return out.reshape(orig_shape)


# speedup vs baseline: 1.4020x; 1.4020x over previous
import functools

import jax
import jax.numpy as jnp
from jax import lax
from jax.experimental import pallas as pl
from jax.experimental.pallas import tpu as pltpu
from jax.experimental.pallas import tpu_sc as plsc

D = 1024
I = 512
SI = 1024
E = 16
EP = 128
T = 4096
K = 2
R = T * K

TMA = 512
NTA = T // TMA
TMG = 512
NTG = R // TMG
NSTEPS = NTG + E - 1
NW = 32
CHUNK = T // NW

NEG = -1e30
WFLOOR = 1e-30


def _lane_val(vec, idx, lanerow):
    return jnp.sum(jnp.where(lanerow == idx, vec, 0))


def _router_shared_body(x32_ref, xb_ref, gate_ref, ws1_ref, ws2_ref,
                        coeff_ref, rank_ref, shared_ref,
                        offs_ref, sm_ref, se_ref, sv_ref, si_ref,
                        cnt_ref):
    t = pl.program_id(0)

    x32 = x32_ref[...]
    logits = jnp.dot(x32, gate_ref[...], preferred_element_type=jnp.float32)
    lane = jax.lax.broadcasted_iota(jnp.int32, logits.shape, 1)
    valid = lane < E
    logits = jnp.where(valid, logits, NEG)
    m = jnp.max(logits, axis=1, keepdims=True)
    p = jnp.exp(logits - m)
    p = jnp.where(valid, p, 0.0)
    p = p / jnp.sum(p, axis=1, keepdims=True)
    p1 = jnp.max(p, axis=1, keepdims=True)
    i1 = jnp.min(jnp.where(p == p1, lane, EP), axis=1, keepdims=True)
    mask1 = lane == i1
    pm = jnp.where(mask1, -1.0, p)
    p2 = jnp.max(pm, axis=1, keepdims=True)
    i2 = jnp.min(jnp.where(pm == p2, lane, EP), axis=1, keepdims=True)
    denom = p1 + p2
    coeff = (jnp.where(mask1, p1, 0.0)
             + jnp.where(lane == i2, p2, 0.0)) / denom
    active = mask1 | (lane == i2)
    coeff = jnp.where(active, jnp.maximum(coeff, WFLOOR), 0.0)
    coeff_ref[...] = coeff[:, :E]

    @pl.when(t == 0)
    def _init_cnt():
        cnt_ref[...] = jnp.zeros((1, EP), jnp.int32)

    act_i = active.astype(jnp.int32)
    s = act_i
    sh = 1
    while sh < TMA:
        z = jnp.zeros((sh, EP), jnp.int32)
        s = s + jnp.concatenate([z, s[:-sh]], axis=0)
        sh *= 2
    excl = s - act_i
    carry = cnt_ref[...]
    rank_ref[...] = (carry + excl)[:, :E]
    new_cnt = carry + jnp.sum(act_i, axis=0, keepdims=True)
    cnt_ref[...] = new_cnt

    xb = xb_ref[...]
    h = jnp.dot(xb, ws1_ref[...], preferred_element_type=jnp.float32)
    act = (jax.nn.silu(h[:, :SI]) * h[:, SI:]).astype(jnp.bfloat16)
    shared_ref[...] = jnp.dot(act, ws2_ref[...],
                              preferred_element_type=jnp.float32)

    @pl.when(t == NTA - 1)
    def _meta():
        lanerow = jax.lax.broadcasted_iota(jnp.int32, (1, EP), 1)
        tot = new_cnt
        o = tot
        shl = 1
        while shl < 32:
            o = o + jnp.concatenate(
                [jnp.zeros((1, shl), jnp.int32), o[:, :-shl]], axis=1)
            shl *= 2
        offs = o - tot
        offs_ref[...] = offs

        e_lo = jnp.zeros((1, EP), jnp.int32)
        e_hi = jnp.zeros((1, EP), jnp.int32)
        for e in range(E):
            off_e = _lane_val(offs, e, lanerow)
            off_e1 = _lane_val(offs, e + 1, lanerow)
            e_lo = e_lo + (off_e1 <= lanerow * TMG).astype(jnp.int32)
            e_hi = e_hi + (off_e < (lanerow + 1) * TMG).astype(jnp.int32)
        e_hi = e_hi - 1
        n = jnp.where(lanerow < NTG, e_hi - e_lo + 1, 0)
        si_ = n
        shl = 1
        while shl < 32:
            si_ = si_ + jnp.concatenate(
                [jnp.zeros((1, shl), jnp.int32), si_[:, :-shl]], axis=1)
            shl *= 2
        start = si_ - n
        total_steps = (_lane_val(start, NTG - 1, lanerow)
                       + _lane_val(n, NTG - 1, lanerow))
        m_of_i = jnp.full((1, EP), -1, jnp.int32)
        for mm in range(NTG):
            s_m = _lane_val(start, mm, lanerow)
            m_of_i = m_of_i + (s_m <= lanerow).astype(jnp.int32)
        m_of_i = jnp.clip(m_of_i, 0, NTG - 1)
        s_at = jnp.zeros((1, EP), jnp.int32)
        elo_at = jnp.zeros((1, EP), jnp.int32)
        for mm in range(NTG):
            cond = m_of_i == mm
            s_at = jnp.where(cond, _lane_val(start, mm, lanerow), s_at)
            elo_at = jnp.where(cond, _lane_val(e_lo, mm, lanerow), elo_at)
        e_of_i = jnp.clip(elo_at + lanerow - s_at, 0, E - 1)
        valid_i = (lanerow < total_steps).astype(jnp.int32)
        init_i = ((lanerow == s_at).astype(jnp.int32)) * valid_i
        sm_ref[...] = m_of_i
        se_ref[...] = e_of_i
        sv_ref[...] = valid_i
        si_ref[...] = init_i


def _gmm_body(sm_ref, se_ref, sv_ref, si_ref, off_ref,
              xs_ref, w1_ref, w2_ref, y_ref):
    i = pl.program_id(0)

    @pl.when(si_ref[i] == 1)
    def _zero():
        y_ref[...] = jnp.zeros_like(y_ref)

    @pl.when(sv_ref[i] == 1)
    def _compute():
        e = se_ref[i]
        lo = off_ref[e]
        hi = off_ref[e + 1]
        xsb = xs_ref[...].astype(jnp.bfloat16)
        h = jnp.dot(xsb, w1_ref[0], preferred_element_type=jnp.float32)
        act = (jax.nn.silu(h[:, :I]) * h[:, I:]).astype(jnp.bfloat16)
        yv = jnp.dot(act, w2_ref[0], preferred_element_type=jnp.float32)
        rows = (sm_ref[i] * TMG
                + jax.lax.broadcasted_iota(jnp.int32, (TMG, 1), 0))
        msk = (rows >= lo) & (rows < hi)
        y_ref[...] += jnp.where(msk, yv, 0.0)


def _make_dispatch():
    mesh = plsc.VectorSubcoreMesh(core_axis_name="c", subcore_axis_name="s")

    @functools.partial(
        pl.kernel,
        out_type=[
            jax.ShapeDtypeStruct((R, D), jnp.float32),
            jax.ShapeDtypeStruct((NW * 2, 64), jnp.int32),
            jax.ShapeDtypeStruct((NW * 2, 64), jnp.int32),
            jax.ShapeDtypeStruct((NW * 2, 64), jnp.float32),
            jax.ShapeDtypeStruct((NW * 2, 64), jnp.float32),
        ],
        mesh=mesh,
        scratch_types=[
            pltpu.VMEM((CHUNK, E), jnp.float32),
            pltpu.VMEM((CHUNK, E), jnp.int32),
            pltpu.VMEM((16,), jnp.int32),
            pltpu.VMEM((2, 64), jnp.int32),
            pltpu.VMEM((2, 64), jnp.int32),
            pltpu.VMEM((2, 64), jnp.float32),
            pltpu.VMEM((2, 64), jnp.float32),
            pltpu.VMEM((64, D), jnp.float32),
            pltpu.SemaphoreType.DMA,
        ],
        compiler_params=pltpu.CompilerParams(
            needs_layout_passes=False, use_tc_tiling_on_sc=True),
    )
    def dispatch(x_hbm, coeff_hbm, rank_hbm, offs_hbm,
                 xs_hbm, d0_hbm, d1_hbm, w0_hbm, w1_hbm,
                 cbuf, rbuf, offs_v, d0buf, d1buf, w0buf, w1buf, xbuf, sem):
        wid = lax.axis_index("s") * 2 + lax.axis_index("c")
        base = wid * CHUNK
        pltpu.sync_copy(coeff_hbm.at[pl.ds(base, CHUNK)], cbuf)
        pltpu.sync_copy(rank_hbm.at[pl.ds(base, CHUNK)], rbuf)
        pltpu.sync_copy(offs_hbm.at[pl.ds(0, 16)], offs_v)
        lanes = lax.iota(jnp.int32, 16)
        offs_vec = offs_v[...]
        for g in range(8):
            d0a = jnp.zeros((16,), jnp.int32)
            d1a = jnp.zeros((16,), jnp.int32)
            w0a = jnp.zeros((16,), jnp.float32)
            w1a = jnp.zeros((16,), jnp.float32)
            for j in range(16):
                tj = g * 16 + j
                v = cbuf[tj, :]
                r = rbuf[tj, :]
                act = v > 0.0
                e0 = jnp.min(jnp.where(act, lanes, E))
                e1 = jnp.min(jnp.where(jnp.logical_and(act, lanes > e0),
                                       lanes, E))
                dd = r + offs_vec
                sel0 = lanes == e0
                sel1 = lanes == e1
                d0s = jnp.sum(jnp.where(sel0, dd, 0))
                d1s = jnp.sum(jnp.where(sel1, dd, 0))
                w0s = jnp.sum(jnp.where(sel0, v, 0.0))
                w1s = jnp.sum(jnp.where(sel1, v, 0.0))
                here = lanes == j
                d0a = jnp.where(here, d0s, d0a)
                d1a = jnp.where(here, d1s, d1a)
                w0a = jnp.where(here, w0s, w0a)
                w1a = jnp.where(here, w1s, w1a)
            sr, hc = g // 4, (g % 4) * 16
            d0buf[sr, pl.ds(hc, 16)] = d0a
            d1buf[sr, pl.ds(hc, 16)] = d1a
            w0buf[sr, pl.ds(hc, 16)] = w0a
            w1buf[sr, pl.ds(hc, 16)] = w1a
        pltpu.sync_copy(d0buf, d0_hbm.at[pl.ds(2 * wid, 2)])
        pltpu.sync_copy(d1buf, d1_hbm.at[pl.ds(2 * wid, 2)])
        pltpu.sync_copy(w0buf, w0_hbm.at[pl.ds(2 * wid, 2)])
        pltpu.sync_copy(w1buf, w1_hbm.at[pl.ds(2 * wid, 2)])
        for sr in range(2):
            pltpu.sync_copy(x_hbm.at[pl.ds(base + 64 * sr, 64)], xbuf)
            pltpu.async_copy(xbuf, xs_hbm.at[d0buf.at[sr]], sem).wait()
            pltpu.async_copy(xbuf, xs_hbm.at[d1buf.at[sr]], sem).wait()

    return dispatch


def _make_combine():
    mesh = plsc.VectorSubcoreMesh(core_axis_name="c", subcore_axis_name="s")

    @functools.partial(
        pl.kernel,
        out_type=jax.ShapeDtypeStruct((T, D), jnp.float32),
        mesh=mesh,
        scratch_types=[
            pltpu.VMEM((2, 64), jnp.int32),
            pltpu.VMEM((2, 64), jnp.int32),
            pltpu.VMEM((2, 64), jnp.float32),
            pltpu.VMEM((2, 64), jnp.float32),
            pltpu.VMEM((32, D), jnp.float32),
            pltpu.VMEM((32, D), jnp.float32),
            pltpu.VMEM((32, D), jnp.float32),
            pltpu.SemaphoreType.DMA,
        ],
        compiler_params=pltpu.CompilerParams(
            needs_layout_passes=False, use_tc_tiling_on_sc=True),
    )
    def combine(y_hbm, d0_hbm, d1_hbm, w0_hbm, w1_hbm, shared_hbm, out_hbm,
                d0v, d1v, w0v, w1v, y0buf, y1buf, sbuf, sem):
        wid = lax.axis_index("s") * 2 + lax.axis_index("c")
        base = wid * CHUNK
        pltpu.sync_copy(d0_hbm.at[pl.ds(2 * wid, 2)], d0v)
        pltpu.sync_copy(d1_hbm.at[pl.ds(2 * wid, 2)], d1v)
        pltpu.sync_copy(w0_hbm.at[pl.ds(2 * wid, 2)], w0v)
        pltpu.sync_copy(w1_hbm.at[pl.ds(2 * wid, 2)], w1v)
        lanes = lax.iota(jnp.int32, 16)
        for q in range(4):
            tb = base + 32 * q
            idx0 = d0v.at[q // 2, pl.ds((q % 2) * 32, 32)]
            idx1 = d1v.at[q // 2, pl.ds((q % 2) * 32, 32)]
            g0 = pltpu.async_copy(y_hbm.at[idx0], y0buf, sem)
            g1 = pltpu.async_copy(y_hbm.at[idx1], y1buf, sem)
            pltpu.sync_copy(shared_hbm.at[pl.ds(tb, 32)], sbuf)
            g0.wait()
            g1.wait()

            def body(j, carry):
                hw = (j // 16) * 16
                w0g = w0v[q // 2, pl.ds((q % 2) * 32 + hw, 16)]
                w1g = w1v[q // 2, pl.ds((q % 2) * 32 + hw, 16)]
                jm = j % 16
                w0s = jnp.sum(jnp.where(lanes == jm, w0g, 0.0))
                w1s = jnp.sum(jnp.where(lanes == jm, w1g, 0.0))
                for kk in range(D // 16):
                    sl = pl.ds(kk * 16, 16)
                    sbuf[j, sl] = (sbuf[j, sl]
                                   + w0s * y0buf[j, sl]
                                   + w1s * y1buf[j, sl])
                return carry

            lax.fori_loop(0, 32, body, 0)
            pltpu.sync_copy(sbuf, out_hbm.at[pl.ds(tb, 32)])

    return combine


def _moe(x, gate_pad, w1b, w2b, ws1b, ws2b, interpret=False):
    xb = x.astype(jnp.bfloat16)

    outs = pl.pallas_call(
        _router_shared_body,
        grid=(NTA,),
        in_specs=[
            pl.BlockSpec((TMA, D), lambda t: (t, 0)),
            pl.BlockSpec((TMA, D), lambda t: (t, 0)),
            pl.BlockSpec((D, EP), lambda t: (0, 0)),
            pl.BlockSpec((D, 2 * SI), lambda t: (0, 0)),
            pl.BlockSpec((SI, D), lambda t: (0, 0)),
        ],
        out_specs=[
            pl.BlockSpec((TMA, E), lambda t: (t, 0)),
            pl.BlockSpec((TMA, E), lambda t: (t, 0)),
            pl.BlockSpec((TMA, D), lambda t: (t, 0)),
            pl.BlockSpec((1, EP), lambda t: (0, 0)),
            pl.BlockSpec((1, EP), lambda t: (0, 0)),
            pl.BlockSpec((1, EP), lambda t: (0, 0)),
            pl.BlockSpec((1, EP), lambda t: (0, 0)),
            pl.BlockSpec((1, EP), lambda t: (0, 0)),
        ],
        out_shape=[
            jax.ShapeDtypeStruct((T, E), jnp.float32),
            jax.ShapeDtypeStruct((T, E), jnp.int32),
            jax.ShapeDtypeStruct((T, D), jnp.float32),
            jax.ShapeDtypeStruct((1, EP), jnp.int32),
            jax.ShapeDtypeStruct((1, EP), jnp.int32),
            jax.ShapeDtypeStruct((1, EP), jnp.int32),
            jax.ShapeDtypeStruct((1, EP), jnp.int32),
            jax.ShapeDtypeStruct((1, EP), jnp.int32),
        ],
        scratch_shapes=[pltpu.VMEM((1, EP), jnp.int32)],
        compiler_params=pltpu.CompilerParams(
            dimension_semantics=("arbitrary",)),
        interpret=interpret,
    )(x, xb, gate_pad, ws1b, ws2b)
    coeff16, rank16, shared, offs, sm, se, sv, si = outs
    offs1 = offs.reshape(EP)
    sm1 = sm.reshape(EP)
    se1 = se.reshape(EP)
    sv1 = sv.reshape(EP)
    si1 = si.reshape(EP)

    dispatch = _make_dispatch()
    xs, d0, d1, w0, w1 = dispatch(x, coeff16, rank16, offs1)

    grid_spec = pltpu.PrefetchScalarGridSpec(
        num_scalar_prefetch=5,
        grid=(NSTEPS,),
        in_specs=[
            pl.BlockSpec((TMG, D), lambda i, sm, se, sv, si, off: (sm[i], 0)),
            pl.BlockSpec((1, D, 2 * I),
                         lambda i, sm, se, sv, si, off: (se[i], 0, 0)),
            pl.BlockSpec((1, I, D),
                         lambda i, sm, se, sv, si, off: (se[i], 0, 0)),
        ],
        out_specs=pl.BlockSpec((TMG, D),
                               lambda i, sm, se, sv, si, off: (sm[i], 0)),
    )
    y = pl.pallas_call(
        _gmm_body,
        grid_spec=grid_spec,
        out_shape=jax.ShapeDtypeStruct((R, D), jnp.float32),
        compiler_params=pltpu.CompilerParams(
            dimension_semantics=("arbitrary",)),
        interpret=interpret,
    )(sm1, se1, sv1, si1, offs1, xs, w1b, w2b)

    combine = _make_combine()
    return combine(y, d0, d1, w0, w1, shared)


def kernel(hidden_states, gate_w, w_gate_up, w_down, ws_gate_up, ws_down):
    orig_shape = hidden_states.shape
    x = hidden_states.reshape(-1, D)
    gate_pad = jnp.pad(gate_w, ((0, 0), (0, EP - E)))
    w1b = w_gate_up.astype(jnp.bfloat16)
    w2b = w_down.astype(jnp.bfloat16)
    ws1b = ws_gate_up.astype(jnp.bfloat16)
    ws2b = ws_down.astype(jnp.bfloat16)
    out = _moe(x, gate_pad, w1b, w2b, ws1b, ws2b)
    return out.reshape(orig_shape)


# speedup vs baseline: 1.8258x; 1.3023x over previous
import functools

import jax
import jax.numpy as jnp
from jax import lax
from jax.experimental import pallas as pl
from jax.experimental.pallas import tpu as pltpu
from jax.experimental.pallas import tpu_sc as plsc

D = 1024
I = 512
SI = 1024
E = 16
EP = 128
T = 4096
K = 2
R = T * K

TMA = 512
NTA = T // TMA
TMG = 512
NTG = R // TMG
NSTEPS = NTG + E - 1
NW = 32
CHUNK = T // NW

NEG = -1e30
WFLOOR = 1e-30


def _lane_val(vec, idx, lanerow):
    return jnp.sum(jnp.where(lanerow == idx, vec, 0))


def _shared_body(x_ref, ws1_ref, ws2_ref, shared_ref):
    xb = x_ref[...].astype(jnp.bfloat16)
    h = jnp.dot(xb, ws1_ref[...].astype(jnp.bfloat16),
                preferred_element_type=jnp.float32)
    act = (jax.nn.silu(h[:, :SI]) * h[:, SI:]).astype(jnp.bfloat16)
    shared_ref[...] = jnp.dot(act, ws2_ref[...].astype(jnp.bfloat16),
                              preferred_element_type=jnp.float32)


def _router_body(x32_ref, gate_ref,
                 coeff_ref, rank_ref,
                 offs_ref, sm_ref, se_ref, sv_ref, si_ref,
                 cnt_ref):
    t = pl.program_id(0)

    x32 = x32_ref[...]
    logits = jnp.dot(x32, gate_ref[...], preferred_element_type=jnp.float32)
    lane = jax.lax.broadcasted_iota(jnp.int32, logits.shape, 1)
    valid = lane < E
    logits = jnp.where(valid, logits, NEG)
    m = jnp.max(logits, axis=1, keepdims=True)
    p = jnp.exp(logits - m)
    p = jnp.where(valid, p, 0.0)
    p = p / jnp.sum(p, axis=1, keepdims=True)
    p1 = jnp.max(p, axis=1, keepdims=True)
    i1 = jnp.min(jnp.where(p == p1, lane, EP), axis=1, keepdims=True)
    mask1 = lane == i1
    pm = jnp.where(mask1, -1.0, p)
    p2 = jnp.max(pm, axis=1, keepdims=True)
    i2 = jnp.min(jnp.where(pm == p2, lane, EP), axis=1, keepdims=True)
    denom = p1 + p2
    coeff = (jnp.where(mask1, p1, 0.0)
             + jnp.where(lane == i2, p2, 0.0)) / denom
    active = mask1 | (lane == i2)
    coeff = jnp.where(active, jnp.maximum(coeff, WFLOOR), 0.0)
    coeff_ref[...] = coeff[:, :E]

    @pl.when(t == 0)
    def _init_cnt():
        cnt_ref[...] = jnp.zeros((1, EP), jnp.int32)

    act_i = active.astype(jnp.int32)
    s = act_i
    sh = 1
    while sh < TMA:
        z = jnp.zeros((sh, EP), jnp.int32)
        s = s + jnp.concatenate([z, s[:-sh]], axis=0)
        sh *= 2
    excl = s - act_i
    carry = cnt_ref[...]
    rank_ref[...] = (carry + excl)[:, :E]
    new_cnt = carry + jnp.sum(act_i, axis=0, keepdims=True)
    cnt_ref[...] = new_cnt

    @pl.when(t == NTA - 1)
    def _meta():
        lanerow = jax.lax.broadcasted_iota(jnp.int32, (1, EP), 1)
        tot = new_cnt
        o = tot
        shl = 1
        while shl < 32:
            o = o + jnp.concatenate(
                [jnp.zeros((1, shl), jnp.int32), o[:, :-shl]], axis=1)
            shl *= 2
        offs = o - tot
        offs_ref[...] = offs

        e_lo = jnp.zeros((1, EP), jnp.int32)
        e_hi = jnp.zeros((1, EP), jnp.int32)
        for e in range(E):
            off_e = _lane_val(offs, e, lanerow)
            off_e1 = _lane_val(offs, e + 1, lanerow)
            e_lo = e_lo + (off_e1 <= lanerow * TMG).astype(jnp.int32)
            e_hi = e_hi + (off_e < (lanerow + 1) * TMG).astype(jnp.int32)
        e_hi = e_hi - 1
        n = jnp.where(lanerow < NTG, e_hi - e_lo + 1, 0)
        si_ = n
        shl = 1
        while shl < 32:
            si_ = si_ + jnp.concatenate(
                [jnp.zeros((1, shl), jnp.int32), si_[:, :-shl]], axis=1)
            shl *= 2
        start = si_ - n
        total_steps = (_lane_val(start, NTG - 1, lanerow)
                       + _lane_val(n, NTG - 1, lanerow))
        m_of_i = jnp.full((1, EP), -1, jnp.int32)
        for mm in range(NTG):
            s_m = _lane_val(start, mm, lanerow)
            m_of_i = m_of_i + (s_m <= lanerow).astype(jnp.int32)
        m_of_i = jnp.clip(m_of_i, 0, NTG - 1)
        s_at = jnp.zeros((1, EP), jnp.int32)
        elo_at = jnp.zeros((1, EP), jnp.int32)
        for mm in range(NTG):
            cond = m_of_i == mm
            s_at = jnp.where(cond, _lane_val(start, mm, lanerow), s_at)
            elo_at = jnp.where(cond, _lane_val(e_lo, mm, lanerow), elo_at)
        e_of_i = jnp.clip(elo_at + lanerow - s_at, 0, E - 1)
        valid_i = (lanerow < total_steps).astype(jnp.int32)
        init_i = ((lanerow == s_at).astype(jnp.int32)) * valid_i
        sm_ref[...] = m_of_i
        se_ref[...] = e_of_i
        sv_ref[...] = valid_i
        si_ref[...] = init_i


def _gmm_body(sm_ref, se_ref, sv_ref, si_ref, off_ref,
              xs_ref, w1_ref, w2_ref, y_ref):
    i = pl.program_id(0)

    @pl.when(si_ref[i] == 1)
    def _zero():
        y_ref[...] = jnp.zeros_like(y_ref)

    @pl.when(sv_ref[i] == 1)
    def _compute():
        e = se_ref[i]
        lo = off_ref[e]
        hi = off_ref[e + 1]
        xsb = xs_ref[...].astype(jnp.bfloat16)
        h = jnp.dot(xsb, w1_ref[0].astype(jnp.bfloat16),
                    preferred_element_type=jnp.float32)
        act = (jax.nn.silu(h[:, :I]) * h[:, I:]).astype(jnp.bfloat16)
        yv = jnp.dot(act, w2_ref[0].astype(jnp.bfloat16),
                     preferred_element_type=jnp.float32)
        rows = (sm_ref[i] * TMG
                + jax.lax.broadcasted_iota(jnp.int32, (TMG, 1), 0))
        msk = (rows >= lo) & (rows < hi)
        y_ref[...] += jnp.where(msk, yv, 0.0)


def _make_dispatch():
    mesh = plsc.VectorSubcoreMesh(core_axis_name="c", subcore_axis_name="s")

    @functools.partial(
        pl.kernel,
        out_type=[
            jax.ShapeDtypeStruct((R, D), jnp.float32),
            jax.ShapeDtypeStruct((NW * 2, 64), jnp.int32),
            jax.ShapeDtypeStruct((NW * 2, 64), jnp.int32),
            jax.ShapeDtypeStruct((NW * 2, 64), jnp.float32),
            jax.ShapeDtypeStruct((NW * 2, 64), jnp.float32),
        ],
        mesh=mesh,
        scratch_types=[
            pltpu.VMEM((CHUNK, E), jnp.float32),
            pltpu.VMEM((CHUNK, E), jnp.int32),
            pltpu.VMEM((16,), jnp.int32),
            pltpu.VMEM((2, 64), jnp.int32),
            pltpu.VMEM((2, 64), jnp.int32),
            pltpu.VMEM((2, 64), jnp.float32),
            pltpu.VMEM((2, 64), jnp.float32),
            pltpu.VMEM((64, D), jnp.float32),
            pltpu.SemaphoreType.DMA,
        ],
        compiler_params=pltpu.CompilerParams(
            needs_layout_passes=False, use_tc_tiling_on_sc=True),
    )
    def dispatch(x_hbm, coeff_hbm, rank_hbm, offs_hbm,
                 xs_hbm, d0_hbm, d1_hbm, w0_hbm, w1_hbm,
                 cbuf, rbuf, offs_v, d0buf, d1buf, w0buf, w1buf, xbuf, sem):
        wid = lax.axis_index("s") * 2 + lax.axis_index("c")
        base = wid * CHUNK
        pltpu.sync_copy(coeff_hbm.at[pl.ds(base, CHUNK)], cbuf)
        pltpu.sync_copy(rank_hbm.at[pl.ds(base, CHUNK)], rbuf)
        pltpu.sync_copy(offs_hbm.at[pl.ds(0, 16)], offs_v)
        lanes = lax.iota(jnp.int32, 16)
        offs_vec = offs_v[...]
        for g in range(8):
            d0a = jnp.zeros((16,), jnp.int32)
            d1a = jnp.zeros((16,), jnp.int32)
            w0a = jnp.zeros((16,), jnp.float32)
            w1a = jnp.zeros((16,), jnp.float32)
            for j in range(16):
                tj = g * 16 + j
                v = cbuf[tj, :]
                r = rbuf[tj, :]
                act = v > 0.0
                e0 = jnp.min(jnp.where(act, lanes, E))
                e1 = jnp.min(jnp.where(jnp.logical_and(act, lanes > e0),
                                       lanes, E))
                dd = r + offs_vec
                sel0 = lanes == e0
                sel1 = lanes == e1
                d0s = jnp.sum(jnp.where(sel0, dd, 0))
                d1s = jnp.sum(jnp.where(sel1, dd, 0))
                w0s = jnp.sum(jnp.where(sel0, v, 0.0))
                w1s = jnp.sum(jnp.where(sel1, v, 0.0))
                here = lanes == j
                d0a = jnp.where(here, d0s, d0a)
                d1a = jnp.where(here, d1s, d1a)
                w0a = jnp.where(here, w0s, w0a)
                w1a = jnp.where(here, w1s, w1a)
            sr, hc = g // 4, (g % 4) * 16
            d0buf[sr, pl.ds(hc, 16)] = d0a
            d1buf[sr, pl.ds(hc, 16)] = d1a
            w0buf[sr, pl.ds(hc, 16)] = w0a
            w1buf[sr, pl.ds(hc, 16)] = w1a
        pltpu.sync_copy(d0buf, d0_hbm.at[pl.ds(2 * wid, 2)])
        pltpu.sync_copy(d1buf, d1_hbm.at[pl.ds(2 * wid, 2)])
        pltpu.sync_copy(w0buf, w0_hbm.at[pl.ds(2 * wid, 2)])
        pltpu.sync_copy(w1buf, w1_hbm.at[pl.ds(2 * wid, 2)])
        for sr in range(2):
            pltpu.sync_copy(x_hbm.at[pl.ds(base + 64 * sr, 64)], xbuf)
            pltpu.async_copy(xbuf, xs_hbm.at[d0buf.at[sr]], sem).wait()
            pltpu.async_copy(xbuf, xs_hbm.at[d1buf.at[sr]], sem).wait()

    return dispatch


def _make_combine():
    mesh = plsc.VectorSubcoreMesh(core_axis_name="c", subcore_axis_name="s")

    @functools.partial(
        pl.kernel,
        out_type=jax.ShapeDtypeStruct((T, D), jnp.float32),
        mesh=mesh,
        scratch_types=[
            pltpu.VMEM((2, 64), jnp.int32),
            pltpu.VMEM((2, 64), jnp.int32),
            pltpu.VMEM((2, 64), jnp.float32),
            pltpu.VMEM((2, 64), jnp.float32),
            pltpu.VMEM((32, D), jnp.float32),
            pltpu.VMEM((32, D), jnp.float32),
            pltpu.VMEM((32, D), jnp.float32),
            pltpu.SemaphoreType.DMA,
        ],
        compiler_params=pltpu.CompilerParams(
            needs_layout_passes=False, use_tc_tiling_on_sc=True),
    )
    def combine(y_hbm, d0_hbm, d1_hbm, w0_hbm, w1_hbm, shared_hbm, out_hbm,
                d0v, d1v, w0v, w1v, y0buf, y1buf, sbuf, sem):
        wid = lax.axis_index("s") * 2 + lax.axis_index("c")
        base = wid * CHUNK
        pltpu.sync_copy(d0_hbm.at[pl.ds(2 * wid, 2)], d0v)
        pltpu.sync_copy(d1_hbm.at[pl.ds(2 * wid, 2)], d1v)
        pltpu.sync_copy(w0_hbm.at[pl.ds(2 * wid, 2)], w0v)
        pltpu.sync_copy(w1_hbm.at[pl.ds(2 * wid, 2)], w1v)
        lanes = lax.iota(jnp.int32, 16)
        for q in range(4):
            tb = base + 32 * q
            idx0 = d0v.at[q // 2, pl.ds((q % 2) * 32, 32)]
            idx1 = d1v.at[q // 2, pl.ds((q % 2) * 32, 32)]
            g0 = pltpu.async_copy(y_hbm.at[idx0], y0buf, sem)
            g1 = pltpu.async_copy(y_hbm.at[idx1], y1buf, sem)
            pltpu.sync_copy(shared_hbm.at[pl.ds(tb, 32)], sbuf)
            g0.wait()
            g1.wait()

            def body(j, carry):
                hw = (j // 16) * 16
                w0g = w0v[q // 2, pl.ds((q % 2) * 32 + hw, 16)]
                w1g = w1v[q // 2, pl.ds((q % 2) * 32 + hw, 16)]
                jm = j % 16
                w0s = jnp.sum(jnp.where(lanes == jm, w0g, 0.0))
                w1s = jnp.sum(jnp.where(lanes == jm, w1g, 0.0))
                for kk in range(D // 16):
                    sl = pl.ds(kk * 16, 16)
                    sbuf[j, sl] = (sbuf[j, sl]
                                   + w0s * y0buf[j, sl]
                                   + w1s * y1buf[j, sl])
                return carry

            lax.fori_loop(0, 32, body, 0)
            pltpu.sync_copy(sbuf, out_hbm.at[pl.ds(tb, 32)])

    return combine


def _moe(x, gate_pad, w1f, w2f, ws1f, ws2f, interpret=False):
    outs = pl.pallas_call(
        _router_body,
        grid=(NTA,),
        in_specs=[
            pl.BlockSpec((TMA, D), lambda t: (t, 0)),
            pl.BlockSpec((D, EP), lambda t: (0, 0)),
        ],
        out_specs=[
            pl.BlockSpec((TMA, E), lambda t: (t, 0)),
            pl.BlockSpec((TMA, E), lambda t: (t, 0)),
            pl.BlockSpec((1, EP), lambda t: (0, 0)),
            pl.BlockSpec((1, EP), lambda t: (0, 0)),
            pl.BlockSpec((1, EP), lambda t: (0, 0)),
            pl.BlockSpec((1, EP), lambda t: (0, 0)),
            pl.BlockSpec((1, EP), lambda t: (0, 0)),
        ],
        out_shape=[
            jax.ShapeDtypeStruct((T, E), jnp.float32),
            jax.ShapeDtypeStruct((T, E), jnp.int32),
            jax.ShapeDtypeStruct((1, EP), jnp.int32),
            jax.ShapeDtypeStruct((1, EP), jnp.int32),
            jax.ShapeDtypeStruct((1, EP), jnp.int32),
            jax.ShapeDtypeStruct((1, EP), jnp.int32),
            jax.ShapeDtypeStruct((1, EP), jnp.int32),
        ],
        scratch_shapes=[pltpu.VMEM((1, EP), jnp.int32)],
        compiler_params=pltpu.CompilerParams(
            dimension_semantics=("arbitrary",)),
        interpret=interpret,
    )(x, gate_pad)
    coeff16, rank16, offs, sm, se, sv, si = outs

    shared = pl.pallas_call(
        _shared_body,
        grid=(NTA,),
        in_specs=[
            pl.BlockSpec((TMA, D), lambda t: (t, 0)),
            pl.BlockSpec((D, 2 * SI), lambda t: (0, 0)),
            pl.BlockSpec((SI, D), lambda t: (0, 0)),
        ],
        out_specs=pl.BlockSpec((TMA, D), lambda t: (t, 0)),
        out_shape=jax.ShapeDtypeStruct((T, D), jnp.float32),
        compiler_params=pltpu.CompilerParams(
            dimension_semantics=("parallel",)),
        interpret=interpret,
    )(x, ws1f, ws2f)
    offs1 = offs.reshape(EP)
    sm1 = sm.reshape(EP)
    se1 = se.reshape(EP)
    sv1 = sv.reshape(EP)
    si1 = si.reshape(EP)

    dispatch = _make_dispatch()
    xs, d0, d1, w0, w1 = dispatch(x, coeff16, rank16, offs1)

    grid_spec = pltpu.PrefetchScalarGridSpec(
        num_scalar_prefetch=5,
        grid=(NSTEPS,),
        in_specs=[
            pl.BlockSpec((TMG, D), lambda i, sm, se, sv, si, off: (sm[i], 0)),
            pl.BlockSpec((1, D, 2 * I),
                         lambda i, sm, se, sv, si, off: (se[i], 0, 0)),
            pl.BlockSpec((1, I, D),
                         lambda i, sm, se, sv, si, off: (se[i], 0, 0)),
        ],
        out_specs=pl.BlockSpec((TMG, D),
                               lambda i, sm, se, sv, si, off: (sm[i], 0)),
    )
    y = pl.pallas_call(
        _gmm_body,
        grid_spec=grid_spec,
        out_shape=jax.ShapeDtypeStruct((R, D), jnp.float32),
        compiler_params=pltpu.CompilerParams(
            dimension_semantics=("arbitrary",)),
        interpret=interpret,
    )(sm1, se1, sv1, si1, offs1, xs, w1f, w2f)

    combine = _make_combine()
    return combine(y, d0, d1, w0, w1, shared)


def kernel(hidden_states, gate_w, w_gate_up, w_down, ws_gate_up, ws_down):
    orig_shape = hidden_states.shape
    x = hidden_states.reshape(-1, D)
    gate_pad = jnp.pad(gate_w, ((0, 0), (0, EP - E)))
    out = _moe(x, gate_pad, w_gate_up, w_down, ws_gate_up, ws_down)
    return out.reshape(orig_shape)


# speedup vs baseline: 1.8955x; 1.0382x over previous
import functools

import jax
import jax.numpy as jnp
from jax import lax
from jax.experimental import pallas as pl
from jax.experimental.pallas import tpu as pltpu
from jax.experimental.pallas import tpu_sc as plsc

D = 1024
I = 512
SI = 1024
E = 16
EP = 128
T = 4096
K = 2
R = T * K

TMA = 512
NTA = T // TMA
TMG = 512
NTG = R // TMG
NSTEPS = NTG + E - 1
NW = 32
CHUNK = T // NW

NEG = -1e30
WFLOOR = 1e-30


def _lane_val(vec, idx, lanerow):
    return jnp.sum(jnp.where(lanerow == idx, vec, 0))


def _shared_body(x_ref, ws1_ref, ws2_ref, shared_ref):
    xb = x_ref[...].astype(jnp.bfloat16)
    h = jnp.dot(xb, ws1_ref[...].astype(jnp.bfloat16),
                preferred_element_type=jnp.float32)
    act = (jax.nn.silu(h[:, :SI]) * h[:, SI:]).astype(jnp.bfloat16)
    shared_ref[...] = jnp.dot(act, ws2_ref[...].astype(jnp.bfloat16),
                              preferred_element_type=jnp.float32)


def _router_body(x32_ref, gate_ref,
                 coeff_ref, rank_ref,
                 offs_ref, sm_ref, se_ref, sv_ref, si_ref,
                 cnt_ref):
    t = pl.program_id(0)

    x32 = x32_ref[...]
    logits = jnp.dot(x32, gate_ref[...], preferred_element_type=jnp.float32)
    lane = jax.lax.broadcasted_iota(jnp.int32, logits.shape, 1)
    valid = lane < E
    logits = jnp.where(valid, logits, NEG)
    m = jnp.max(logits, axis=1, keepdims=True)
    p = jnp.exp(logits - m)
    p = jnp.where(valid, p, 0.0)
    p = p / jnp.sum(p, axis=1, keepdims=True)
    p1 = jnp.max(p, axis=1, keepdims=True)
    i1 = jnp.min(jnp.where(p == p1, lane, EP), axis=1, keepdims=True)
    mask1 = lane == i1
    pm = jnp.where(mask1, -1.0, p)
    p2 = jnp.max(pm, axis=1, keepdims=True)
    i2 = jnp.min(jnp.where(pm == p2, lane, EP), axis=1, keepdims=True)
    denom = p1 + p2
    coeff = (jnp.where(mask1, p1, 0.0)
             + jnp.where(lane == i2, p2, 0.0)) / denom
    active = mask1 | (lane == i2)
    coeff = jnp.where(active, jnp.maximum(coeff, WFLOOR), 0.0)
    coeff_ref[...] = coeff[:, :E]

    @pl.when(t == 0)
    def _init_cnt():
        cnt_ref[...] = jnp.zeros((1, EP), jnp.int32)

    act_i = active.astype(jnp.int32)
    s = act_i
    sh = 1
    while sh < TMA:
        z = jnp.zeros((sh, EP), jnp.int32)
        s = s + jnp.concatenate([z, s[:-sh]], axis=0)
        sh *= 2
    excl = s - act_i
    carry = cnt_ref[...]
    rank_ref[...] = (carry + excl)[:, :E]
    new_cnt = carry + jnp.sum(act_i, axis=0, keepdims=True)
    cnt_ref[...] = new_cnt

    @pl.when(t == NTA - 1)
    def _meta():
        lanerow = jax.lax.broadcasted_iota(jnp.int32, (1, EP), 1)
        tot = new_cnt
        o = tot
        shl = 1
        while shl < 32:
            o = o + jnp.concatenate(
                [jnp.zeros((1, shl), jnp.int32), o[:, :-shl]], axis=1)
            shl *= 2
        offs = o - tot
        offs_ref[...] = offs

        e_lo = jnp.zeros((1, EP), jnp.int32)
        e_hi = jnp.zeros((1, EP), jnp.int32)
        for e in range(E):
            off_e = _lane_val(offs, e, lanerow)
            off_e1 = _lane_val(offs, e + 1, lanerow)
            e_lo = e_lo + (off_e1 <= lanerow * TMG).astype(jnp.int32)
            e_hi = e_hi + (off_e < (lanerow + 1) * TMG).astype(jnp.int32)
        e_hi = e_hi - 1
        n = jnp.where(lanerow < NTG, e_hi - e_lo + 1, 0)
        si_ = n
        shl = 1
        while shl < 32:
            si_ = si_ + jnp.concatenate(
                [jnp.zeros((1, shl), jnp.int32), si_[:, :-shl]], axis=1)
            shl *= 2
        start = si_ - n
        total_steps = (_lane_val(start, NTG - 1, lanerow)
                       + _lane_val(n, NTG - 1, lanerow))
        m_of_i = jnp.full((1, EP), -1, jnp.int32)
        for mm in range(NTG):
            s_m = _lane_val(start, mm, lanerow)
            m_of_i = m_of_i + (s_m <= lanerow).astype(jnp.int32)
        m_of_i = jnp.clip(m_of_i, 0, NTG - 1)
        s_at = jnp.zeros((1, EP), jnp.int32)
        elo_at = jnp.zeros((1, EP), jnp.int32)
        for mm in range(NTG):
            cond = m_of_i == mm
            s_at = jnp.where(cond, _lane_val(start, mm, lanerow), s_at)
            elo_at = jnp.where(cond, _lane_val(e_lo, mm, lanerow), elo_at)
        e_of_i = jnp.clip(elo_at + lanerow - s_at, 0, E - 1)
        valid_i = (lanerow < total_steps).astype(jnp.int32)
        init_i = ((lanerow == s_at).astype(jnp.int32)) * valid_i
        sm_ref[...] = m_of_i
        se_ref[...] = e_of_i
        sv_ref[...] = valid_i
        si_ref[...] = init_i


def _gmm_body(sm_ref, se_ref, sv_ref, si_ref, off_ref,
              xs_ref, w1_ref, w2_ref, y_ref):
    i = pl.program_id(0)

    @pl.when(si_ref[i] == 1)
    def _zero():
        y_ref[...] = jnp.zeros_like(y_ref)

    @pl.when(sv_ref[i] == 1)
    def _compute():
        e = se_ref[i]
        lo = off_ref[e]
        hi = off_ref[e + 1]
        xsb = xs_ref[...].astype(jnp.bfloat16)
        h = jnp.dot(xsb, w1_ref[0].astype(jnp.bfloat16),
                    preferred_element_type=jnp.float32)
        act = (jax.nn.silu(h[:, :I]) * h[:, I:]).astype(jnp.bfloat16)
        yv = jnp.dot(act, w2_ref[0].astype(jnp.bfloat16),
                     preferred_element_type=jnp.float32)
        rows = (sm_ref[i] * TMG
                + jax.lax.broadcasted_iota(jnp.int32, (TMG, 1), 0))
        msk = (rows >= lo) & (rows < hi)
        y_ref[...] += jnp.where(msk, yv, 0.0)


def _make_dispatch():
    mesh = plsc.VectorSubcoreMesh(core_axis_name="c", subcore_axis_name="s")

    @functools.partial(
        pl.kernel,
        out_type=[
            jax.ShapeDtypeStruct((R, D), jnp.float32),
            jax.ShapeDtypeStruct((NW * 8, 16), jnp.int32),
            jax.ShapeDtypeStruct((NW * 8, 16), jnp.int32),
            jax.ShapeDtypeStruct((NW * 8, 16), jnp.float32),
            jax.ShapeDtypeStruct((NW * 8, 16), jnp.float32),
        ],
        mesh=mesh,
        scratch_types=[
            pltpu.VMEM((CHUNK, E), jnp.float32),
            pltpu.VMEM((CHUNK, E), jnp.int32),
            pltpu.VMEM((16,), jnp.int32),
            pltpu.VMEM((8, 16), jnp.int32),
            pltpu.VMEM((8, 16), jnp.int32),
            pltpu.VMEM((8, 16), jnp.float32),
            pltpu.VMEM((8, 16), jnp.float32),
            pltpu.VMEM((2, 16, D), jnp.float32),
            pltpu.SemaphoreType.DMA,
            pltpu.SemaphoreType.DMA,
            pltpu.SemaphoreType.DMA,
            pltpu.SemaphoreType.DMA,
        ],
        compiler_params=pltpu.CompilerParams(
            needs_layout_passes=False, use_tc_tiling_on_sc=True),
    )
    def dispatch(x_hbm, coeff_hbm, rank_hbm, offs_hbm,
                 xs_hbm, d0_hbm, d1_hbm, w0_hbm, w1_hbm,
                 cbuf, rbuf, offs_v, d0buf, d1buf, w0buf, w1buf, xbuf,
                 si0, si1, ss0, ss1):
        wid = lax.axis_index("s") * 2 + lax.axis_index("c")
        base = wid * CHUNK
        s_in = (si0, si1)
        s_sc = (ss0, ss1)
        in_h = [pltpu.async_copy(x_hbm.at[pl.ds(base + 16 * q, 16)],
                                 xbuf.at[q % 2], s_in[q % 2])
                for q in range(2)]
        pltpu.sync_copy(coeff_hbm.at[pl.ds(base, CHUNK)], cbuf)
        pltpu.sync_copy(rank_hbm.at[pl.ds(base, CHUNK)], rbuf)
        pltpu.sync_copy(offs_hbm.at[pl.ds(0, 16)], offs_v)
        lanes = lax.iota(jnp.int32, 16)
        offs_vec = offs_v[...]
        for g in range(8):
            d0a = jnp.zeros((16,), jnp.int32)
            d1a = jnp.zeros((16,), jnp.int32)
            w0a = jnp.zeros((16,), jnp.float32)
            w1a = jnp.zeros((16,), jnp.float32)
            for j in range(16):
                tj = g * 16 + j
                v = cbuf[tj, :]
                r = rbuf[tj, :]
                act = v > 0.0
                e0 = jnp.min(jnp.where(act, lanes, E))
                e1 = jnp.min(jnp.where(jnp.logical_and(act, lanes > e0),
                                       lanes, E))
                dd = r + offs_vec
                sel0 = lanes == e0
                sel1 = lanes == e1
                d0s = jnp.sum(jnp.where(sel0, dd, 0))
                d1s = jnp.sum(jnp.where(sel1, dd, 0))
                w0s = jnp.sum(jnp.where(sel0, v, 0.0))
                w1s = jnp.sum(jnp.where(sel1, v, 0.0))
                here = lanes == j
                d0a = jnp.where(here, d0s, d0a)
                d1a = jnp.where(here, d1s, d1a)
                w0a = jnp.where(here, w0s, w0a)
                w1a = jnp.where(here, w1s, w1a)
            d0buf[g, :] = d0a
            d1buf[g, :] = d1a
            w0buf[g, :] = w0a
            w1buf[g, :] = w1a
        pltpu.sync_copy(d0buf, d0_hbm.at[pl.ds(8 * wid, 8)])
        pltpu.sync_copy(d1buf, d1_hbm.at[pl.ds(8 * wid, 8)])
        pltpu.sync_copy(w0buf, w0_hbm.at[pl.ds(8 * wid, 8)])
        pltpu.sync_copy(w1buf, w1_hbm.at[pl.ds(8 * wid, 8)])
        tail = [None, None]
        for q in range(8):
            b = q % 2
            in_h[b].wait()
            h0 = pltpu.async_copy(xbuf.at[b], xs_hbm.at[d0buf.at[q]], s_sc[b])
            h1 = pltpu.async_copy(xbuf.at[b], xs_hbm.at[d1buf.at[q]], s_sc[b])
            if q + 2 < 8:
                h0.wait()
                h1.wait()
                in_h[b] = pltpu.async_copy(
                    x_hbm.at[pl.ds(base + 16 * (q + 2), 16)],
                    xbuf.at[b], s_in[b])
            else:
                tail[b] = (h0, h1)
        for pair in tail:
            pair[0].wait()
            pair[1].wait()

    return dispatch


def _make_combine():
    mesh = plsc.VectorSubcoreMesh(core_axis_name="c", subcore_axis_name="s")

    @functools.partial(
        pl.kernel,
        out_type=jax.ShapeDtypeStruct((T, D), jnp.float32),
        mesh=mesh,
        scratch_types=[
            pltpu.VMEM((8, 16), jnp.int32),
            pltpu.VMEM((8, 16), jnp.int32),
            pltpu.VMEM((8, 16), jnp.float32),
            pltpu.VMEM((8, 16), jnp.float32),
            pltpu.VMEM((2, 16, D), jnp.float32),
            pltpu.VMEM((2, 16, D), jnp.float32),
            pltpu.VMEM((2, 16, D), jnp.float32),
            pltpu.SemaphoreType.DMA,
            pltpu.SemaphoreType.DMA,
            pltpu.SemaphoreType.DMA,
            pltpu.SemaphoreType.DMA,
        ],
        compiler_params=pltpu.CompilerParams(
            needs_layout_passes=False, use_tc_tiling_on_sc=True),
    )
    def combine(y_hbm, d0_hbm, d1_hbm, w0_hbm, w1_hbm, shared_hbm, out_hbm,
                d0v, d1v, w0v, w1v, y0buf, y1buf, sbuf,
                si0, si1, so0, so1):
        wid = lax.axis_index("s") * 2 + lax.axis_index("c")
        base = wid * CHUNK
        s_in = (si0, si1)
        s_out = (so0, so1)
        pltpu.sync_copy(d0_hbm.at[pl.ds(8 * wid, 8)], d0v)
        pltpu.sync_copy(d1_hbm.at[pl.ds(8 * wid, 8)], d1v)
        pltpu.sync_copy(w0_hbm.at[pl.ds(8 * wid, 8)], w0v)
        pltpu.sync_copy(w1_hbm.at[pl.ds(8 * wid, 8)], w1v)
        lanes = lax.iota(jnp.int32, 16)

        def stage_in(q):
            b = q % 2
            tb = base + 16 * q
            return (
                pltpu.async_copy(y_hbm.at[d0v.at[q]], y0buf.at[b], s_in[b]),
                pltpu.async_copy(y_hbm.at[d1v.at[q]], y1buf.at[b], s_in[b]),
                pltpu.async_copy(shared_hbm.at[pl.ds(tb, 16)],
                                 sbuf.at[b], s_in[b]),
            )

        in_h = [stage_in(0), stage_in(1)]
        tail = [None, None]
        for q in range(8):
            b = q % 2
            for h in in_h[b]:
                h.wait()
            w0g = w0v[q, :]
            w1g = w1v[q, :]

            def body(j, carry):
                w0s = jnp.sum(jnp.where(lanes == j, w0g, 0.0))
                w1s = jnp.sum(jnp.where(lanes == j, w1g, 0.0))
                for kk in range(D // 16):
                    sl = pl.ds(kk * 16, 16)
                    sbuf[b, j, sl] = (sbuf[b, j, sl]
                                      + w0s * y0buf[b, j, sl]
                                      + w1s * y1buf[b, j, sl])
                return carry

            lax.fori_loop(0, 16, body, 0)
            oh = pltpu.async_copy(sbuf.at[b],
                                  out_hbm.at[pl.ds(base + 16 * q, 16)],
                                  s_out[b])
            if q + 2 < 8:
                oh.wait()
                in_h[b] = stage_in(q + 2)
            else:
                tail[b] = oh
        tail[0].wait()
        tail[1].wait()

    return combine


def _moe(x, gate_pad, w1f, w2f, ws1f, ws2f, interpret=False):
    outs = pl.pallas_call(
        _router_body,
        grid=(NTA,),
        in_specs=[
            pl.BlockSpec((TMA, D), lambda t: (t, 0)),
            pl.BlockSpec((D, EP), lambda t: (0, 0)),
        ],
        out_specs=[
            pl.BlockSpec((TMA, E), lambda t: (t, 0)),
            pl.BlockSpec((TMA, E), lambda t: (t, 0)),
            pl.BlockSpec((1, EP), lambda t: (0, 0)),
            pl.BlockSpec((1, EP), lambda t: (0, 0)),
            pl.BlockSpec((1, EP), lambda t: (0, 0)),
            pl.BlockSpec((1, EP), lambda t: (0, 0)),
            pl.BlockSpec((1, EP), lambda t: (0, 0)),
        ],
        out_shape=[
            jax.ShapeDtypeStruct((T, E), jnp.float32),
            jax.ShapeDtypeStruct((T, E), jnp.int32),
            jax.ShapeDtypeStruct((1, EP), jnp.int32),
            jax.ShapeDtypeStruct((1, EP), jnp.int32),
            jax.ShapeDtypeStruct((1, EP), jnp.int32),
            jax.ShapeDtypeStruct((1, EP), jnp.int32),
            jax.ShapeDtypeStruct((1, EP), jnp.int32),
        ],
        scratch_shapes=[pltpu.VMEM((1, EP), jnp.int32)],
        compiler_params=pltpu.CompilerParams(
            dimension_semantics=("arbitrary",)),
        interpret=interpret,
    )(x, gate_pad)
    coeff16, rank16, offs, sm, se, sv, si = outs

    shared = pl.pallas_call(
        _shared_body,
        grid=(NTA,),
        in_specs=[
            pl.BlockSpec((TMA, D), lambda t: (t, 0)),
            pl.BlockSpec((D, 2 * SI), lambda t: (0, 0)),
            pl.BlockSpec((SI, D), lambda t: (0, 0)),
        ],
        out_specs=pl.BlockSpec((TMA, D), lambda t: (t, 0)),
        out_shape=jax.ShapeDtypeStruct((T, D), jnp.float32),
        compiler_params=pltpu.CompilerParams(
            dimension_semantics=("parallel",)),
        interpret=interpret,
    )(x, ws1f, ws2f)
    offs1 = offs.reshape(EP)
    sm1 = sm.reshape(EP)
    se1 = se.reshape(EP)
    sv1 = sv.reshape(EP)
    si1 = si.reshape(EP)

    dispatch = _make_dispatch()
    xs, d0, d1, w0, w1 = dispatch(x, coeff16, rank16, offs1)

    grid_spec = pltpu.PrefetchScalarGridSpec(
        num_scalar_prefetch=5,
        grid=(NSTEPS,),
        in_specs=[
            pl.BlockSpec((TMG, D), lambda i, sm, se, sv, si, off: (sm[i], 0)),
            pl.BlockSpec((1, D, 2 * I),
                         lambda i, sm, se, sv, si, off: (se[i], 0, 0)),
            pl.BlockSpec((1, I, D),
                         lambda i, sm, se, sv, si, off: (se[i], 0, 0)),
        ],
        out_specs=pl.BlockSpec((TMG, D),
                               lambda i, sm, se, sv, si, off: (sm[i], 0)),
    )
    y = pl.pallas_call(
        _gmm_body,
        grid_spec=grid_spec,
        out_shape=jax.ShapeDtypeStruct((R, D), jnp.float32),
        compiler_params=pltpu.CompilerParams(
            dimension_semantics=("arbitrary",)),
        interpret=interpret,
    )(sm1, se1, sv1, si1, offs1, xs, w1f, w2f)

    combine = _make_combine()
    return combine(y, d0, d1, w0, w1, shared)


def kernel(hidden_states, gate_w, w_gate_up, w_down, ws_gate_up, ws_down):
    orig_shape = hidden_states.shape
    x = hidden_states.reshape(-1, D)
    gate_pad = jnp.pad(gate_w, ((0, 0), (0, EP - E)))
    out = _moe(x, gate_pad, w_gate_up, w_down, ws_gate_up, ws_down)
    return out.reshape(orig_shape)
